# Initial kernel scaffold; baseline (speedup 1.0000x reference)
import jax
import jax.numpy as jnp
from jax.experimental import pallas as pl


def kernel(x, edge_index, arg0, arg1, W0, b0, W1, b1, W2, b2):
    raise NotImplementedError("write your pallas kernel here")


# speedup vs baseline: 8.3669x; 8.3669x over previous
import functools

import jax
import jax.numpy as jnp
from jax import lax
from jax.experimental import pallas as pl
from jax.experimental.pallas import tpu as pltpu
from jax.experimental.pallas import tpu_sc as plsc

NC = 2
NS = 16
K = 128
BR = 256


def _round_up(a, b):
    return (a + b - 1) // b * b


def _sc_mesh():
    return plsc.VectorSubcoreMesh(core_axis_name="c", subcore_axis_name="s")


@functools.cache
def _make_deg_kernel(N_pad, E_pad):
    C = E_pad // (NC * NS * K)
    R = N_pad // NS

    def body(dst_hbm, consts_hbm, out_hbm, didx, cbuf, acc):
        cid = lax.axis_index("c")
        sid = lax.axis_index("s")
        pltpu.sync_copy(consts_hbm.at[0], cbuf)

        @pl.loop(0, R // K)
        def _(j):
            pltpu.sync_copy(cbuf, acc.at[pl.ds(sid * R + j * K, K)])

        pltpu.sync_copy(consts_hbm.at[1], cbuf)
        plsc.subcore_barrier()

        base = (cid * NS + sid) * C * K

        @pl.loop(0, C)
        def _(j):
            pltpu.sync_copy(dst_hbm.at[pl.ds(base + j * K, K)], didx)
            pltpu.sync_copy(cbuf, acc.at[didx], add=True)

        plsc.subcore_barrier()

        @pl.loop(0, R // K)
        def _(j):
            r0 = sid * R + j * K
            pltpu.sync_copy(acc.at[pl.ds(r0, K)], cbuf)
            pltpu.sync_copy(cbuf, out_hbm.at[pl.ds(cid * N_pad + r0, K)])

    return pl.kernel(
        body,
        out_type=jax.ShapeDtypeStruct((NC * N_pad, 128), jnp.float32),
        mesh=_sc_mesh(),
        scratch_types=[
            pltpu.VMEM((K,), jnp.int32),
            pltpu.VMEM((K, 128), jnp.float32),
            pltpu.VMEM_SHARED((N_pad, 128), jnp.float32),
        ],
    )


@functools.cache
def _make_edge_kernel(N_pad, E_pad, H):
    C = E_pad // (NC * NS * K)
    R = N_pad // NS

    def body(ys_hbm, src_hbm, dst_hbm, zeros_hbm, out_hbm, sidx, didx, rows, acc):
        cid = lax.axis_index("c")
        sid = lax.axis_index("s")
        pltpu.sync_copy(zeros_hbm, rows)

        @pl.loop(0, R // K)
        def _(j):
            pltpu.sync_copy(rows, acc.at[pl.ds(sid * R + j * K, K)])

        plsc.subcore_barrier()

        base = (cid * NS + sid) * C * K

        @pl.loop(0, C)
        def _(j):
            pltpu.sync_copy(src_hbm.at[pl.ds(base + j * K, K)], sidx)
            pltpu.sync_copy(dst_hbm.at[pl.ds(base + j * K, K)], didx)
            pltpu.sync_copy(ys_hbm.at[sidx], rows)
            pltpu.sync_copy(rows, acc.at[didx], add=True)

        plsc.subcore_barrier()

        @pl.loop(0, R // K)
        def _(j):
            r0 = sid * R + j * K
            pltpu.sync_copy(acc.at[pl.ds(r0, K)], rows)
            pltpu.sync_copy(rows, out_hbm.at[pl.ds(cid * N_pad + r0, K)])

    return pl.kernel(
        body,
        out_type=jax.ShapeDtypeStruct((NC * N_pad, H), jnp.float32),
        mesh=_sc_mesh(),
        scratch_types=[
            pltpu.VMEM((K,), jnp.int32),
            pltpu.VMEM((K,), jnp.int32),
            pltpu.VMEM((K, H), jnp.float32),
            pltpu.VMEM_SHARED((N_pad, H), jnp.float32),
        ],
    )


def _dinv_block(degp_ref, i, N):
    deg = degp_ref[0, :, 0:1] + degp_ref[1, :, 0:1] + 1.0
    row = i * BR + lax.broadcasted_iota(jnp.int32, (BR, 1), 0)
    return lax.rsqrt(deg) * (row < N).astype(jnp.float32)


@functools.cache
def _make_tc_first(N, N_pad, D, H):
    def body(x_ref, w_ref, degp_ref, o_ref):
        dinv = _dinv_block(degp_ref, pl.program_id(0), N)
        o_ref[...] = dinv * jnp.dot(x_ref[...], w_ref[...],
                                    preferred_element_type=jnp.float32)

    return pl.pallas_call(
        body,
        grid=(N_pad // BR,),
        in_specs=[
            pl.BlockSpec((BR, D), lambda i: (i, 0)),
            pl.BlockSpec((D, H), lambda i: (0, 0)),
            pl.BlockSpec((2, BR, 128), lambda i: (0, i, 0)),
        ],
        out_specs=pl.BlockSpec((BR, H), lambda i: (i, 0)),
        out_shape=jax.ShapeDtypeStruct((N_pad, H), jnp.float32),
    )


@functools.cache
def _make_tc_mid(N, N_pad, H):
    def body(p_ref, ys_ref, w_ref, b_ref, degp_ref, o_ref):
        dinv = _dinv_block(degp_ref, pl.program_id(0), N)
        agg = p_ref[0] + p_ref[1] + ys_ref[...]
        h = jnp.maximum(dinv * agg + b_ref[...], 0.0)
        o_ref[...] = dinv * jnp.dot(h, w_ref[...],
                                    preferred_element_type=jnp.float32)

    return pl.pallas_call(
        body,
        grid=(N_pad // BR,),
        in_specs=[
            pl.BlockSpec((2, BR, H), lambda i: (0, i, 0)),
            pl.BlockSpec((BR, H), lambda i: (i, 0)),
            pl.BlockSpec((H, H), lambda i: (0, 0)),
            pl.BlockSpec((1, H), lambda i: (0, 0)),
            pl.BlockSpec((2, BR, 128), lambda i: (0, i, 0)),
        ],
        out_specs=pl.BlockSpec((BR, H), lambda i: (i, 0)),
        out_shape=jax.ShapeDtypeStruct((N_pad, H), jnp.float32),
    )


@functools.cache
def _make_tc_final(N, N_pad, H):
    def body(p_ref, ys_ref, b_ref, degp_ref, o_ref):
        dinv = _dinv_block(degp_ref, pl.program_id(0), N)
        agg = p_ref[0] + p_ref[1] + ys_ref[...]
        o_ref[...] = dinv * agg + b_ref[...]

    return pl.pallas_call(
        body,
        grid=(N_pad // BR,),
        in_specs=[
            pl.BlockSpec((2, BR, H), lambda i: (0, i, 0)),
            pl.BlockSpec((BR, H), lambda i: (i, 0)),
            pl.BlockSpec((1, H), lambda i: (0, 0)),
            pl.BlockSpec((2, BR, 128), lambda i: (0, i, 0)),
        ],
        out_specs=pl.BlockSpec((BR, H), lambda i: (i, 0)),
        out_shape=jax.ShapeDtypeStruct((N_pad, H), jnp.float32),
    )


def kernel(x, edge_index, arg0, arg1, W0, b0, W1, b1, W2, b2):
    N, D = x.shape
    H = W0.shape[1]
    E = edge_index.shape[1]
    NW = NC * NS
    N_pad = _round_up(N + 1, NS * K)
    E_pad = _round_up(E, NW * K)

    src = edge_index[0].astype(jnp.int32)
    dst = edge_index[1].astype(jnp.int32)
    if E_pad > E:
        padv = jnp.full((E_pad - E,), N, jnp.int32)
        src = jnp.concatenate([src, padv])
        dst = jnp.concatenate([dst, padv])
    x_p = jnp.pad(x.astype(jnp.float32), ((0, N_pad - N), (0, 0)))
    consts = jnp.stack([jnp.zeros((K, 128), jnp.float32),
                        jnp.ones((K, 128), jnp.float32)])
    zeros_kh = jnp.zeros((K, H), jnp.float32)

    if False:
        ones_e = jnp.ones((E_pad,), jnp.float32)
        degc = jax.ops.segment_sum(ones_e[:E], dst[:E], num_segments=N_pad)
        degp = jnp.broadcast_to(
            jnp.stack([degc, jnp.zeros_like(degc)])[:, :, None], (NC, N_pad, 128))

        def edge(ys, src, dst, zeros_kh):
            agg = jax.ops.segment_sum(ys[src], dst, num_segments=N_pad)
            return jnp.concatenate([agg, jnp.zeros_like(agg)], axis=0)
    else:
        degp = _make_deg_kernel(N_pad, E_pad)(dst, consts).reshape(NC, N_pad, 128)
        edge = _make_edge_kernel(N_pad, E_pad, H)
    mid = _make_tc_mid(N, N_pad, H)

    ys = _make_tc_first(N, N_pad, D, H)(x_p, W0.astype(jnp.float32), degp)
    p = edge(ys, src, dst, zeros_kh).reshape(NC, N_pad, H)
    ys = mid(p, ys, W1.astype(jnp.float32), b0.reshape(1, H), degp)
    p = edge(ys, src, dst, zeros_kh).reshape(NC, N_pad, H)
    ys = mid(p, ys, W2.astype(jnp.float32), b1.reshape(1, H), degp)
    p = edge(ys, src, dst, zeros_kh).reshape(NC, N_pad, H)
    out = _make_tc_final(N, N_pad, H)(p, ys, b2.reshape(1, H), degp)

    return out[:N], arg1
